# baseline (speedup 1.0000x reference)
import jax
import jax.numpy as jnp
from jax.experimental import pallas as pl
from jax.experimental.pallas import tpu as pltpu


_TM = 512
_NBUF = 4


def _outer(x_ref, s_hbm, o_hbm):
    def body(s_ref, o_ref):
        o_ref[...] = jnp.dot(s_ref[...], x_ref[...],
                             preferred_element_type=jnp.float32)

    m = s_hbm.shape[0]
    pltpu.emit_pipeline(
        body,
        grid=(m // _TM,),
        in_specs=[
            pl.BlockSpec((_TM, s_hbm.shape[1]), lambda i: (i, 0),
                         pipeline_mode=pl.Buffered(buffer_count=_NBUF)),
        ],
        out_specs=[
            pl.BlockSpec((_TM, o_hbm.shape[1]), lambda i: (i, 0)),
        ],
    )(s_hbm, o_hbm)


def kernel(x_coarse, interp_matrix):
    m, k = interp_matrix.shape
    n = x_coarse.shape[1]
    return pl.pallas_call(
        _outer,
        in_specs=[
            pl.BlockSpec(memory_space=pltpu.MemorySpace.VMEM),
            pl.BlockSpec(memory_space=pl.ANY),
        ],
        out_specs=pl.BlockSpec(memory_space=pl.ANY),
        out_shape=jax.ShapeDtypeStruct((m, n), jnp.float32),
    )(x_coarse, interp_matrix)


# speedup vs baseline: 1.0450x; 1.0450x over previous
import jax
import jax.numpy as jnp
from jax.experimental import pallas as pl
from jax.experimental.pallas import tpu as pltpu


_TM = 512


def _interp_tile(s_ref, x_ref, o_ref):
    o_ref[...] = jnp.dot(s_ref[...], x_ref[...],
                         preferred_element_type=jnp.float32)


def kernel(x_coarse, interp_matrix):
    m, k = interp_matrix.shape
    n = x_coarse.shape[1]
    return pl.pallas_call(
        _interp_tile,
        grid=(m // _TM,),
        in_specs=[
            pl.BlockSpec((_TM, k), lambda i: (i, 0)),
            pl.BlockSpec(memory_space=pltpu.MemorySpace.VMEM),
        ],
        out_specs=pl.BlockSpec((_TM, n), lambda i: (i, 0)),
        out_shape=jax.ShapeDtypeStruct((m, n), jnp.float32),
        compiler_params=pltpu.CompilerParams(
            skip_device_barrier=True,
        ),
    )(interp_matrix, x_coarse)
